# baseline (speedup 1.0000x reference)
import functools

import jax
import jax.numpy as jnp
from jax import lax
from jax.experimental import pallas as pl
from jax.experimental.pallas import tpu as pltpu
from jax.experimental.pallas import tpu_sc as plsc

_NCODES = 8192
_D = 256
_M = 32768
_BM = 1024
_BN = 1024
_MT = _M // _BM
_NSUB = _NCODES // _BN
_COMMIT = 0.25


def _argmin_body(z_ref, e_ref, idx_ref, loss_ref, e2_ref, iw_ref):
    i = pl.program_id(0)

    @pl.when(i == 0)
    def _precompute_e2():
        for j in range(_NSUB):
            eb = e_ref[pl.ds(j * _BN, _BN), :]
            e2_ref[0, pl.ds(j * _BN, _BN)] = jnp.sum(eb * eb, axis=1)
        rowi = lax.broadcasted_iota(jnp.int32, (_BN, 128), 0).astype(jnp.float32)
        coli = lax.broadcasted_iota(jnp.int32, (_BN, 128), 1)
        iw_ref[...] = jnp.where(
            coli == 0, rowi, jnp.where(coli == 1, 1.0, 0.0)
        )

    zb = z_ref[...]
    z2 = jnp.sum(zb * zb, axis=1, keepdims=True)
    zm2 = zb * (-2.0)
    iota = lax.broadcasted_iota(jnp.int32, (_BM, _BN), 1).astype(jnp.float32)

    run_min = jnp.full((_BM, 1), jnp.inf, jnp.float32)
    run_idx = jnp.zeros((_BM, 1), jnp.float32)
    for j in range(_NSUB):
        eb = e_ref[pl.ds(j * _BN, _BN), :]
        s2 = lax.dot_general(
            zm2, eb,
            dimension_numbers=(((1,), (1,)), ((), ())),
            preferred_element_type=jnp.float32,
        )
        d = (z2 + s2) + e2_ref[0, pl.ds(j * _BN, _BN)][None, :]
        bmin = jnp.min(d, axis=1, keepdims=True)
        onehot = jnp.where(d == bmin, 1.0, 0.0)
        agg = lax.dot_general(
            onehot, iw_ref[...],
            dimension_numbers=(((1,), (0,)), ((), ())),
            preferred_element_type=jnp.float32,
        )
        idx_sum = agg[:, 0:1]
        count = agg[:, 1:2]
        local = lax.cond(
            jnp.max(count) > 1.0,
            lambda: jnp.min(
                jnp.where(d == bmin, iota, float(_NCODES)),
                axis=1, keepdims=True,
            ),
            lambda: idx_sum,
        )
        barg = local + float(j * _BN)
        upd = bmin < run_min
        run_idx = jnp.where(upd, barg, run_idx)
        run_min = jnp.where(upd, bmin, run_min)

    idx_ref[...] = run_idx[:, 0].astype(jnp.int32)

    @pl.when(i == 0)
    def _init():
        loss_ref[...] = jnp.zeros((1, 1), jnp.float32)

    loss_ref[...] = loss_ref[...] + jnp.sum(run_min)

    @pl.when(i == _MT - 1)
    def _finish():
        loss_ref[...] = loss_ref[...] * (_COMMIT / (_M * _D))


_argmin_call = pl.pallas_call(
    _argmin_body,
    grid=(_MT,),
    in_specs=[
        pl.BlockSpec((_BM, _D), lambda i: (i, 0)),
        pl.BlockSpec((_NCODES, _D), lambda i: (0, 0)),
    ],
    out_specs=[
        pl.BlockSpec((_BM,), lambda i: (i,)),
        pl.BlockSpec((1, 1), lambda i: (0, 0)),
    ],
    out_shape=[
        jax.ShapeDtypeStruct((_M,), jnp.int32),
        jax.ShapeDtypeStruct((1, 1), jnp.float32),
    ],
    scratch_shapes=[
        pltpu.VMEM((1, _NCODES), jnp.float32),
        pltpu.VMEM((_BN, 128), jnp.float32),
    ],
)


_NC = 2
_NS = 16
_NW = _NC * _NS
_BPW = _M // _NW
_CH = 128
_NCHUNK = _BPW // _CH


@functools.cache
def _gather_codes():
    @functools.partial(
        pl.kernel,
        mesh=plsc.VectorSubcoreMesh(
            core_axis_name="c", subcore_axis_name="s", num_cores=_NC
        ),
        out_type=jax.ShapeDtypeStruct((_M, _D), jnp.float32),
        scratch_types=[
            pltpu.VMEM((_BPW,), jnp.int32),
            pltpu.VMEM((_CH, _D), jnp.float32),
            pltpu.VMEM((_CH, _D), jnp.float32),
            pltpu.SemaphoreType.DMA,
            pltpu.SemaphoreType.DMA,
            pltpu.SemaphoreType.DMA,
            pltpu.SemaphoreType.DMA,
        ],
    )
    def gather_k(table_hbm, idx_hbm, out_hbm, idx_v, rows0, rows1,
                 gsem0, gsem1, wsem0, wsem1):
        wid = lax.axis_index("s") * _NC + lax.axis_index("c")
        base = wid * _BPW
        pltpu.sync_copy(idx_hbm.at[pl.ds(base, _BPW)], idx_v)
        rows = (rows0, rows1)
        gsems = (gsem0, gsem1)
        wsems = (wsem0, wsem1)
        gd = [None, None]
        wd = [None, None]
        for c in range(_NCHUNK):
            b = c & 1
            if c >= 2:
                wd[b].wait()
            gd[b] = pltpu.async_copy(
                table_hbm.at[idx_v.at[pl.ds(c * _CH, _CH)]], rows[b], gsems[b]
            )
            if c >= 1:
                pb = (c - 1) & 1
                gd[pb].wait()
                wd[pb] = pltpu.async_copy(
                    rows[pb], out_hbm.at[pl.ds(base + (c - 1) * _CH, _CH)],
                    wsems[pb],
                )
        lb = (_NCHUNK - 1) & 1
        gd[lb].wait()
        wd[lb] = pltpu.async_copy(
            rows[lb], out_hbm.at[pl.ds(base + (_NCHUNK - 1) * _CH, _CH)],
            wsems[lb],
        )
        wd[0].wait()
        wd[1].wait()

    return gather_k


def kernel(z, embedding):
    orig_dtype = z.dtype
    z32 = z.astype(jnp.float32)
    z_flat = z32.reshape(-1, _D)
    indices, loss = _argmin_call(z_flat, embedding)
    z_q_flat = _gather_codes()(embedding, indices)
    z_q = z_q_flat.reshape(z32.shape).astype(orig_dtype)
    return (z_q, indices, loss[0, 0])


# speedup vs baseline: 2.1092x; 2.1092x over previous
import functools

import jax
import jax.numpy as jnp
from jax import lax
from jax.experimental import pallas as pl
from jax.experimental.pallas import tpu as pltpu
from jax.experimental.pallas import tpu_sc as plsc

_NCODES = 8192
_D = 256
_M = 32768
_BM = 2048
_BN = 1024
_MT = _M // _BM
_NSUB = _NCODES // _BN
_COMMIT = 0.25


def _argmin_body(z_ref, e_ref, idx_ref, loss_ref, e2_ref):
    i = pl.program_id(0)

    @pl.when(i == 0)
    def _precompute_e2():
        for j in range(_NSUB):
            eb = e_ref[pl.ds(j * _BN, _BN), :]
            e2_ref[0, pl.ds(j * _BN, _BN)] = jnp.sum(eb * eb, axis=1)

    zb = z_ref[...]
    z2 = jnp.sum(zb * zb, axis=1, keepdims=True)
    zm2 = zb * (-2.0)
    iota = lax.broadcasted_iota(jnp.int32, (_BM, _BN), 1).astype(jnp.float32)

    run_min = jnp.full((_BM, 1), jnp.inf, jnp.float32)
    run_idx = jnp.zeros((_BM, 1), jnp.float32)
    for j in range(_NSUB):
        eb = e_ref[pl.ds(j * _BN, _BN), :]
        s2 = lax.dot_general(
            zm2, eb,
            dimension_numbers=(((1,), (1,)), ((), ())),
            preferred_element_type=jnp.float32,
        )
        d = (z2 + s2) + e2_ref[0, pl.ds(j * _BN, _BN)][None, :]
        bmin = jnp.min(d, axis=1, keepdims=True)
        local = jnp.min(
            jnp.where(d == bmin, iota, float(_NCODES)), axis=1, keepdims=True
        )
        barg = local + float(j * _BN)
        upd = bmin < run_min
        run_idx = jnp.where(upd, barg, run_idx)
        run_min = jnp.where(upd, bmin, run_min)

    idx_ref[...] = run_idx[:, 0].astype(jnp.int32)

    @pl.when(i == 0)
    def _init():
        loss_ref[...] = jnp.zeros((1, 1), jnp.float32)

    loss_ref[...] = loss_ref[...] + jnp.sum(run_min)

    @pl.when(i == _MT - 1)
    def _finish():
        loss_ref[...] = loss_ref[...] * (_COMMIT / (_M * _D))


_argmin_call = pl.pallas_call(
    _argmin_body,
    grid=(_MT,),
    in_specs=[
        pl.BlockSpec((_BM, _D), lambda i: (i, 0)),
        pl.BlockSpec((_NCODES, _D), lambda i: (0, 0)),
    ],
    out_specs=[
        pl.BlockSpec((_BM,), lambda i: (i,)),
        pl.BlockSpec((1, 1), lambda i: (0, 0)),
    ],
    out_shape=[
        jax.ShapeDtypeStruct((_M,), jnp.int32),
        jax.ShapeDtypeStruct((1, 1), jnp.float32),
    ],
    scratch_shapes=[pltpu.VMEM((1, _NCODES), jnp.float32)],
)


_NC = 2
_NS = 16
_NW = _NC * _NS
_BPW = _M // _NW
_CH = 128
_NCHUNK = _BPW // _CH


@functools.cache
def _gather_codes():
    @functools.partial(
        pl.kernel,
        mesh=plsc.VectorSubcoreMesh(
            core_axis_name="c", subcore_axis_name="s", num_cores=_NC
        ),
        out_type=jax.ShapeDtypeStruct((_M, _D), jnp.float32),
        scratch_types=[
            pltpu.VMEM((_BPW,), jnp.int32),
            pltpu.VMEM((_CH, _D), jnp.float32),
            pltpu.VMEM((_CH, _D), jnp.float32),
            pltpu.SemaphoreType.DMA,
            pltpu.SemaphoreType.DMA,
            pltpu.SemaphoreType.DMA,
            pltpu.SemaphoreType.DMA,
        ],
    )
    def gather_k(table_hbm, idx_hbm, out_hbm, idx_v, rows0, rows1,
                 gsem0, gsem1, wsem0, wsem1):
        wid = lax.axis_index("s") * _NC + lax.axis_index("c")
        base = wid * _BPW
        pltpu.sync_copy(idx_hbm.at[pl.ds(base, _BPW)], idx_v)
        rows = (rows0, rows1)
        gsems = (gsem0, gsem1)
        wsems = (wsem0, wsem1)
        gd = [None, None]
        wd = [None, None]
        for c in range(_NCHUNK):
            b = c & 1
            if c >= 2:
                wd[b].wait()
            gd[b] = pltpu.async_copy(
                table_hbm.at[idx_v.at[pl.ds(c * _CH, _CH)]], rows[b], gsems[b]
            )
            if c >= 1:
                pb = (c - 1) & 1
                gd[pb].wait()
                wd[pb] = pltpu.async_copy(
                    rows[pb], out_hbm.at[pl.ds(base + (c - 1) * _CH, _CH)],
                    wsems[pb],
                )
        lb = (_NCHUNK - 1) & 1
        gd[lb].wait()
        wd[lb] = pltpu.async_copy(
            rows[lb], out_hbm.at[pl.ds(base + (_NCHUNK - 1) * _CH, _CH)],
            wsems[lb],
        )
        wd[0].wait()
        wd[1].wait()

    return gather_k


def kernel(z, embedding):
    orig_dtype = z.dtype
    z32 = z.astype(jnp.float32)
    z_flat = z32.reshape(-1, _D)
    indices, loss = _argmin_call(z_flat, embedding)
    z_q_flat = _gather_codes()(embedding, indices)
    z_q = z_q_flat.reshape(z32.shape).astype(orig_dtype)
    return (z_q, indices, loss[0, 0])
